# baseline (device time: 99868 ns/iter reference)
import jax
import jax.numpy as jnp
from jax import lax
from jax.experimental import pallas as pl
from jax.experimental.pallas import tpu as pltpu

N_DEV = 16
S = 4
N_MSG = S * (N_DEV // 2 - 1) + S // 2


def kernel(x):
    m_per, n = x.shape
    m_sub = m_per // S

    def body(x_ref, out_ref, send_r, recv_r, send_l, recv_l):
        my_pos = lax.axis_index("i")
        left = lax.rem(my_pos - 1 + N_DEV, N_DEV)
        right = lax.rem(my_pos + 1, N_DEV)

        barrier_sem = pltpu.get_barrier_semaphore()
        pl.semaphore_signal(
            barrier_sem, inc=1, device_id=(left,),
            device_id_type=pl.DeviceIdType.MESH,
        )
        pl.semaphore_signal(
            barrier_sem, inc=1, device_id=(right,),
            device_id_type=pl.DeviceIdType.MESH,
        )
        pl.semaphore_wait(barrier_sem, 2)

        def out_sub(origin, s):
            return out_ref.at[pl.ds(origin * m_per + s * m_sub, m_sub), :]

        out_r, in_r, out_l, in_l = [], [], [], []
        for k in range(N_MSG):
            j = k // S
            sr = k % S
            sl = S - 1 - sr
            o_out_r = lax.rem(my_pos - j + N_DEV, N_DEV)
            o_in_r = lax.rem(my_pos - j - 1 + N_DEV, N_DEV)
            o_out_l = lax.rem(my_pos + j, N_DEV)
            o_in_l = lax.rem(my_pos + j + 1, N_DEV)
            src_r = (x_ref.at[pl.ds(sr * m_sub, m_sub), :] if j == 0
                     else out_sub(o_out_r, sr))
            src_l = (x_ref.at[pl.ds(sl * m_sub, m_sub), :] if j == 0
                     else out_sub(o_out_l, sl))
            out_r.append(pltpu.make_async_remote_copy(
                src_ref=src_r, dst_ref=out_sub(o_out_r, sr),
                send_sem=send_r.at[k], recv_sem=recv_r.at[k],
                device_id=(right,), device_id_type=pl.DeviceIdType.MESH,
            ))
            in_r.append(pltpu.make_async_remote_copy(
                src_ref=out_sub(o_in_r, sr), dst_ref=out_sub(o_in_r, sr),
                send_sem=send_r.at[k], recv_sem=recv_r.at[k],
                device_id=(right,), device_id_type=pl.DeviceIdType.MESH,
            ))
            out_l.append(pltpu.make_async_remote_copy(
                src_ref=src_l, dst_ref=out_sub(o_out_l, sl),
                send_sem=send_l.at[k], recv_sem=recv_l.at[k],
                device_id=(left,), device_id_type=pl.DeviceIdType.MESH,
            ))
            in_l.append(pltpu.make_async_remote_copy(
                src_ref=out_sub(o_in_l, sl), dst_ref=out_sub(o_in_l, sl),
                send_sem=send_l.at[k], recv_sem=recv_l.at[k],
                device_id=(left,), device_id_type=pl.DeviceIdType.MESH,
            ))

        for k in range(S):
            out_r[k].start()
            out_l[k].start()
        out_ref[pl.ds(my_pos * m_per, m_per), :] = x_ref[:, :]

        for k in range(S, N_MSG):
            in_r[k - S].wait_recv()
            out_r[k].start()
            in_l[k - S].wait_recv()
            out_l[k].start()

        for k in range(N_MSG - S, N_MSG):
            in_r[k].wait_recv()
            in_l[k].wait_recv()
        for k in range(N_MSG):
            out_r[k].wait_send()
            out_l[k].wait_send()

    return pl.pallas_call(
        body,
        out_shape=jax.ShapeDtypeStruct((N_DEV * m_per, n), x.dtype),
        in_specs=[pl.BlockSpec(memory_space=pltpu.VMEM)],
        out_specs=pl.BlockSpec(memory_space=pltpu.VMEM),
        scratch_shapes=[
            pltpu.SemaphoreType.DMA((N_MSG,)),
            pltpu.SemaphoreType.DMA((N_MSG,)),
            pltpu.SemaphoreType.DMA((N_MSG,)),
            pltpu.SemaphoreType.DMA((N_MSG,)),
        ],
        compiler_params=pltpu.CompilerParams(collective_id=0),
    )(x)


# device time: 99320 ns/iter; 1.0055x vs baseline; 1.0055x over previous
import jax
import jax.numpy as jnp
from jax import lax
from jax.experimental import pallas as pl
from jax.experimental.pallas import tpu as pltpu

N_DEV = 16
S = 2
N_MSG = S * (N_DEV // 2 - 1) + S // 2


def kernel(x):
    m_per, n = x.shape
    m_sub = m_per // S

    def body(x_ref, out_ref, send_r, recv_r, send_l, recv_l):
        my_pos = lax.axis_index("i")
        left = lax.rem(my_pos - 1 + N_DEV, N_DEV)
        right = lax.rem(my_pos + 1, N_DEV)

        barrier_sem = pltpu.get_barrier_semaphore()
        pl.semaphore_signal(
            barrier_sem, inc=1, device_id=(left,),
            device_id_type=pl.DeviceIdType.MESH,
        )
        pl.semaphore_signal(
            barrier_sem, inc=1, device_id=(right,),
            device_id_type=pl.DeviceIdType.MESH,
        )
        pl.semaphore_wait(barrier_sem, 2)

        def out_sub(origin, s):
            return out_ref.at[pl.ds(origin * m_per + s * m_sub, m_sub), :]

        out_r, in_r, out_l, in_l = [], [], [], []
        for k in range(N_MSG):
            j = k // S
            sr = k % S
            sl = S - 1 - sr
            o_out_r = lax.rem(my_pos - j + N_DEV, N_DEV)
            o_in_r = lax.rem(my_pos - j - 1 + N_DEV, N_DEV)
            o_out_l = lax.rem(my_pos + j, N_DEV)
            o_in_l = lax.rem(my_pos + j + 1, N_DEV)
            src_r = (x_ref.at[pl.ds(sr * m_sub, m_sub), :] if j == 0
                     else out_sub(o_out_r, sr))
            src_l = (x_ref.at[pl.ds(sl * m_sub, m_sub), :] if j == 0
                     else out_sub(o_out_l, sl))
            out_r.append(pltpu.make_async_remote_copy(
                src_ref=src_r, dst_ref=out_sub(o_out_r, sr),
                send_sem=send_r.at[k], recv_sem=recv_r.at[k],
                device_id=(right,), device_id_type=pl.DeviceIdType.MESH,
            ))
            in_r.append(pltpu.make_async_remote_copy(
                src_ref=out_sub(o_in_r, sr), dst_ref=out_sub(o_in_r, sr),
                send_sem=send_r.at[k], recv_sem=recv_r.at[k],
                device_id=(right,), device_id_type=pl.DeviceIdType.MESH,
            ))
            out_l.append(pltpu.make_async_remote_copy(
                src_ref=src_l, dst_ref=out_sub(o_out_l, sl),
                send_sem=send_l.at[k], recv_sem=recv_l.at[k],
                device_id=(left,), device_id_type=pl.DeviceIdType.MESH,
            ))
            in_l.append(pltpu.make_async_remote_copy(
                src_ref=out_sub(o_in_l, sl), dst_ref=out_sub(o_in_l, sl),
                send_sem=send_l.at[k], recv_sem=recv_l.at[k],
                device_id=(left,), device_id_type=pl.DeviceIdType.MESH,
            ))

        for k in range(S):
            out_r[k].start()
            out_l[k].start()
        out_ref[pl.ds(my_pos * m_per, m_per), :] = x_ref[:, :]

        for k in range(S, N_MSG):
            in_r[k - S].wait_recv()
            out_r[k].start()
            in_l[k - S].wait_recv()
            out_l[k].start()

        for k in range(N_MSG - S, N_MSG):
            in_r[k].wait_recv()
            in_l[k].wait_recv()
        for k in range(N_MSG):
            out_r[k].wait_send()
            out_l[k].wait_send()

    return pl.pallas_call(
        body,
        out_shape=jax.ShapeDtypeStruct((N_DEV * m_per, n), x.dtype),
        in_specs=[pl.BlockSpec(memory_space=pltpu.VMEM)],
        out_specs=pl.BlockSpec(memory_space=pltpu.VMEM),
        scratch_shapes=[
            pltpu.SemaphoreType.DMA((N_MSG,)),
            pltpu.SemaphoreType.DMA((N_MSG,)),
            pltpu.SemaphoreType.DMA((N_MSG,)),
            pltpu.SemaphoreType.DMA((N_MSG,)),
        ],
        compiler_params=pltpu.CompilerParams(collective_id=0),
    )(x)
